# baseline (speedup 1.0000x reference)
import functools

import jax
import jax.numpy as jnp
from jax.experimental import pallas as pl


def _avg_pool3(x):
    s = jax.lax.reduce_window(x, 0.0, jax.lax.add, (1, 1, 3, 3), (1, 1, 1, 1),
                              ((0, 0), (0, 0), (1, 1), (1, 1)))
    return s / 9.0


def _fused_body(x0_ref, x1_ref, x2_ref, w0_ref, w1_ref, w2_ref, ct_ref,
                cent_ref, centers_ref, out_ref):
    x0 = x0_ref[0].astype(jnp.bfloat16)
    x1 = x1_ref[0].astype(jnp.bfloat16)
    x2 = x2_ref[0].astype(jnp.bfloat16)
    e = (jnp.dot(w0_ref[...], x0, preferred_element_type=jnp.float32)
         + jnp.dot(w1_ref[...], x1, preferred_element_type=jnp.float32)
         + jnp.dot(w2_ref[...], x2, preferred_element_type=jnp.float32)
         + ct_ref[...])
    feats = jnp.sum(e * e, axis=0)[:, None]
    eb = e.astype(jnp.bfloat16)
    prod = jax.lax.dot_general(eb, cent_ref[...],
                               (((0,), (0,)), ((), ())),
                               preferred_element_type=jnp.float32)
    d2 = feats + centers_ref[...] - 2.0 * prod

    iota = jax.lax.broadcasted_iota(jnp.int32, d2.shape, 1)
    cur = d2
    mins = []
    for _ in range(3):
        mins.append(jnp.min(cur, axis=1))
        am = jnp.argmin(cur, axis=1)
        cur = jnp.where(iota == am[:, None], jnp.inf, cur)
    d0 = jnp.sqrt(jnp.maximum(mins[0], 1e-12))
    d1 = jnp.sqrt(jnp.maximum(mins[1], 1e-12))
    d2s = jnp.sqrt(jnp.maximum(mins[2], 1e-12))
    sm0 = 1.0 / (1.0 + jnp.exp(d0 - d1) + jnp.exp(d0 - d2s))
    out_ref[0, 0] = sm0 * d0


@functools.partial(jax.jit, static_argnums=())
def kernel(p0, p1, p2, W, bconv, centroids):
    b = p0.shape[0]
    h, w = p0.shape[2], p0.shape[3]
    hw = h * w
    c = centroids.shape[0]
    n = centroids.shape[1]
    c0, c1, c2 = p0.shape[1], p1.shape[1], p2.shape[1]

    bm = 640 if hw > 640 else hw
    nblk = -(-hw // bm)

    def prep(p):
        a = _avg_pool3(p)
        if a.shape[2] != h:
            a = jax.image.resize(a, (b, a.shape[1], h, w), method='bilinear')
        return a.reshape(b, a.shape[1], hw)

    x0, x1, x2 = prep(p0), prep(p1), prep(p2)

    xx = (jnp.arange(w, dtype=jnp.float32) / (w - 1)) * 2.0 - 1.0
    yy = (jnp.arange(h, dtype=jnp.float32) / (h - 1)) * 2.0 - 1.0
    grid_x = jnp.tile(xx, h)
    grid_y = jnp.repeat(yy, w)
    ct = (W[:, c, None] * grid_x[None, :] + W[:, c + 1, None] * grid_y[None, :]
          + bconv[:, None])
    w0 = W[:, :c0].astype(jnp.bfloat16)
    w1 = W[:, c0:c0 + c1].astype(jnp.bfloat16)
    w2 = W[:, c0 + c1:c].astype(jnp.bfloat16)
    centb = centroids.astype(jnp.bfloat16)
    centers = jnp.sum(centroids * centroids, axis=0, keepdims=True)

    score = pl.pallas_call(
        _fused_body,
        grid=(b, nblk),
        in_specs=[
            pl.BlockSpec((1, c0, bm), lambda i, j: (i, 0, j)),
            pl.BlockSpec((1, c1, bm), lambda i, j: (i, 0, j)),
            pl.BlockSpec((1, c2, bm), lambda i, j: (i, 0, j)),
            pl.BlockSpec((c, c0), lambda i, j: (0, 0)),
            pl.BlockSpec((c, c1), lambda i, j: (0, 0)),
            pl.BlockSpec((c, c2), lambda i, j: (0, 0)),
            pl.BlockSpec((c, bm), lambda i, j: (0, j)),
            pl.BlockSpec((c, n), lambda i, j: (0, 0)),
            pl.BlockSpec((1, n), lambda i, j: (0, 0)),
        ],
        out_specs=pl.BlockSpec((1, 1, bm), lambda i, j: (i * nblk + j, 0, 0)),
        out_shape=jax.ShapeDtypeStruct((b * nblk, 1, bm), jnp.float32),
    )(x0, x1, x2, w0, w1, w2, ct, centb, centers)

    return score.reshape(b, nblk * bm)[:, :hw].reshape(b, 1, h, w)


# speedup vs baseline: 1.4761x; 1.4761x over previous
import functools

import numpy as np

import jax
import jax.numpy as jnp
from jax.experimental import pallas as pl


def _lin_mat(n_out, n_in):
    if n_out == n_in:
        return np.eye(n_in, dtype=np.float64)
    src = (np.arange(n_out) + 0.5) * (n_in / n_out) - 0.5
    i0 = np.floor(src).astype(np.int64)
    f = src - i0
    m = np.zeros((n_out, n_in), dtype=np.float64)
    rows = np.arange(n_out)
    m[rows, np.clip(i0, 0, n_in - 1)] += 1.0 - f
    m[rows, np.clip(i0 + 1, 0, n_in - 1)] += f
    return m


def _pool_mat(n):
    m = np.zeros((n, n), dtype=np.float64)
    idx = np.arange(n)
    for d in (-1, 0, 1):
        j = idx + d
        ok = (j >= 0) & (j < n)
        m[idx[ok], j[ok]] = 1.0
    return m


def _prep_mat(h_out, w_out, h_in, w_in):
    mh = _lin_mat(h_out, h_in) @ _pool_mat(h_in)
    mw = _lin_mat(w_out, w_in) @ _pool_mat(w_in)
    m = np.kron(mh, mw) / 9.0
    return m.T.astype(np.float32)


def _fused_body(p0_ref, p1_ref, p2_ref, m0_ref, m1_ref, m2_ref,
                w0_ref, w1_ref, w2_ref, wxy_ref, bc_ref, gxy_ref,
                cent_ref, centers_ref, out_ref):
    xp0 = jnp.dot(p0_ref[0].astype(jnp.bfloat16), m0_ref[...],
                  preferred_element_type=jnp.float32).astype(jnp.bfloat16)
    xp1 = jnp.dot(p1_ref[0].astype(jnp.bfloat16), m1_ref[...],
                  preferred_element_type=jnp.float32).astype(jnp.bfloat16)
    xp2 = jnp.dot(p2_ref[0].astype(jnp.bfloat16), m2_ref[...],
                  preferred_element_type=jnp.float32).astype(jnp.bfloat16)
    ct = (wxy_ref[:, 0:1] * gxy_ref[0:1, :] + wxy_ref[:, 1:2] * gxy_ref[1:2, :]
          + bc_ref[...])
    e = (jnp.dot(w0_ref[...], xp0, preferred_element_type=jnp.float32)
         + jnp.dot(w1_ref[...], xp1, preferred_element_type=jnp.float32)
         + jnp.dot(w2_ref[...], xp2, preferred_element_type=jnp.float32)
         + ct)
    feats = jnp.sum(e * e, axis=0)[:, None]
    eb = e.astype(jnp.bfloat16)
    prod = jax.lax.dot_general(eb, cent_ref[...],
                               (((0,), (0,)), ((), ())),
                               preferred_element_type=jnp.float32)
    d2 = feats + centers_ref[...] - 2.0 * prod

    iota = jax.lax.broadcasted_iota(jnp.int32, d2.shape, 1)
    cur = d2
    mins = []
    for _ in range(3):
        mins.append(jnp.min(cur, axis=1))
        am = jnp.argmin(cur, axis=1)
        cur = jnp.where(iota == am[:, None], jnp.inf, cur)
    d0 = jnp.sqrt(jnp.maximum(mins[0], 1e-12))
    d1 = jnp.sqrt(jnp.maximum(mins[1], 1e-12))
    d2s = jnp.sqrt(jnp.maximum(mins[2], 1e-12))
    sm0 = 1.0 / (1.0 + jnp.exp(d0 - d1) + jnp.exp(d0 - d2s))
    out_ref[0, 0] = sm0 * d0


@functools.partial(jax.jit, static_argnums=())
def kernel(p0, p1, p2, W, bconv, centroids):
    b = p0.shape[0]
    h, w = p0.shape[2], p0.shape[3]
    hw = h * w
    c = centroids.shape[0]
    n = centroids.shape[1]
    c0, c1, c2 = p0.shape[1], p1.shape[1], p2.shape[1]
    hw0 = p0.shape[2] * p0.shape[3]
    hw1 = p1.shape[2] * p1.shape[3]
    hw2 = p2.shape[2] * p2.shape[3]

    bm = 640 if hw > 640 else hw
    nblk = -(-hw // bm)

    m0 = jnp.asarray(_prep_mat(h, w, p0.shape[2], p0.shape[3]), jnp.bfloat16)
    m1 = jnp.asarray(_prep_mat(h, w, p1.shape[2], p1.shape[3]), jnp.bfloat16)
    m2 = jnp.asarray(_prep_mat(h, w, p2.shape[2], p2.shape[3]), jnp.bfloat16)

    xx = (jnp.arange(w, dtype=jnp.float32) / (w - 1)) * 2.0 - 1.0
    yy = (jnp.arange(h, dtype=jnp.float32) / (h - 1)) * 2.0 - 1.0
    gxy = jnp.stack([jnp.tile(xx, h), jnp.repeat(yy, w)])
    w0 = W[:, :c0].astype(jnp.bfloat16)
    w1 = W[:, c0:c0 + c1].astype(jnp.bfloat16)
    w2 = W[:, c0 + c1:c].astype(jnp.bfloat16)
    wxy = W[:, c:c + 2]
    bc = bconv[:, None]
    centb = centroids.astype(jnp.bfloat16)
    centers = jnp.sum(centroids * centroids, axis=0, keepdims=True)

    score = pl.pallas_call(
        _fused_body,
        grid=(b, nblk),
        in_specs=[
            pl.BlockSpec((1, c0, hw0), lambda i, j: (i, 0, 0)),
            pl.BlockSpec((1, c1, hw1), lambda i, j: (i, 0, 0)),
            pl.BlockSpec((1, c2, hw2), lambda i, j: (i, 0, 0)),
            pl.BlockSpec((hw0, bm), lambda i, j: (0, j)),
            pl.BlockSpec((hw1, bm), lambda i, j: (0, j)),
            pl.BlockSpec((hw2, bm), lambda i, j: (0, j)),
            pl.BlockSpec((c, c0), lambda i, j: (0, 0)),
            pl.BlockSpec((c, c1), lambda i, j: (0, 0)),
            pl.BlockSpec((c, c2), lambda i, j: (0, 0)),
            pl.BlockSpec((c, 2), lambda i, j: (0, 0)),
            pl.BlockSpec((c, 1), lambda i, j: (0, 0)),
            pl.BlockSpec((2, bm), lambda i, j: (0, j)),
            pl.BlockSpec((c, n), lambda i, j: (0, 0)),
            pl.BlockSpec((1, n), lambda i, j: (0, 0)),
        ],
        out_specs=pl.BlockSpec((1, 1, bm), lambda i, j: (i * nblk + j, 0, 0)),
        out_shape=jax.ShapeDtypeStruct((b * nblk, 1, bm), jnp.float32),
    )(p0.reshape(b, c0, hw0), p1.reshape(b, c1, hw1), p2.reshape(b, c2, hw2),
      m0, m1, m2, w0, w1, w2, wxy, bc, gxy, centb, centers)

    return score.reshape(b, nblk * bm)[:, :hw].reshape(b, 1, h, w)


# speedup vs baseline: 1.7441x; 1.1816x over previous
import functools

import numpy as np

import jax
import jax.numpy as jnp
from jax.experimental import pallas as pl
from jax.experimental.pallas import tpu as pltpu


def _lin_mat(n_out, n_in):
    if n_out == n_in:
        return np.eye(n_in, dtype=np.float64)
    src = (np.arange(n_out) + 0.5) * (n_in / n_out) - 0.5
    i0 = np.floor(src).astype(np.int64)
    f = src - i0
    m = np.zeros((n_out, n_in), dtype=np.float64)
    rows = np.arange(n_out)
    m[rows, np.clip(i0, 0, n_in - 1)] += 1.0 - f
    m[rows, np.clip(i0 + 1, 0, n_in - 1)] += f
    return m


def _pool_mat(n):
    m = np.zeros((n, n), dtype=np.float64)
    idx = np.arange(n)
    for d in (-1, 0, 1):
        j = idx + d
        ok = (j >= 0) & (j < n)
        m[idx[ok], j[ok]] = 1.0
    return m


def _prep_mat(h_out, w_out, h_in, w_in):
    mh = _lin_mat(h_out, h_in) @ _pool_mat(h_in)
    mw = _lin_mat(w_out, w_in) @ _pool_mat(w_in)
    m = np.kron(mh, mw) / 9.0
    return m.T.astype(np.float32)


def _fused_body(p0_ref, p1_ref, p2_ref, m0_ref, m1_ref, m2_ref,
                w0_ref, w1_ref, w2_ref, wxy_ref, bc_ref, gxy_ref,
                cent_ref, centers_ref, out_ref, z2_ref):
    @pl.when(pl.program_id(1) == 0)
    def _():
        z2_ref[...] = jnp.dot(w2_ref[...], p2_ref[0].astype(jnp.bfloat16),
                              preferred_element_type=jnp.float32
                              ).astype(jnp.bfloat16)
    xp0 = jnp.dot(p0_ref[0].astype(jnp.bfloat16), m0_ref[...],
                  preferred_element_type=jnp.float32).astype(jnp.bfloat16)
    xp1 = jnp.dot(p1_ref[0].astype(jnp.bfloat16), m1_ref[...],
                  preferred_element_type=jnp.float32).astype(jnp.bfloat16)
    ct = (wxy_ref[:, 0:1] * gxy_ref[0:1, :] + wxy_ref[:, 1:2] * gxy_ref[1:2, :]
          + bc_ref[...])
    e = (jnp.dot(w0_ref[...], xp0, preferred_element_type=jnp.float32)
         + jnp.dot(w1_ref[...], xp1, preferred_element_type=jnp.float32)
         + jnp.dot(z2_ref[...], m2_ref[...], preferred_element_type=jnp.float32)
         + ct)
    feats = jnp.sum(e * e, axis=0)[:, None]
    eb = e.astype(jnp.bfloat16)
    prod = jax.lax.dot_general(eb, cent_ref[...],
                               (((0,), (0,)), ((), ())),
                               preferred_element_type=jnp.float32)
    d2 = feats + centers_ref[...] - 2.0 * prod

    cur = d2
    mins = []
    for _ in range(3):
        mins.append(jnp.min(cur, axis=1))
        cur = jnp.where(cur <= mins[-1][:, None], jnp.inf, cur)
    d0 = jnp.sqrt(jnp.maximum(mins[0], 1e-12))
    d1 = jnp.sqrt(jnp.maximum(mins[1], 1e-12))
    d2s = jnp.sqrt(jnp.maximum(mins[2], 1e-12))
    sm0 = 1.0 / (1.0 + jnp.exp(d0 - d1) + jnp.exp(d0 - d2s))
    out_ref[0, 0] = sm0 * d0


@functools.partial(jax.jit, static_argnums=())
def kernel(p0, p1, p2, W, bconv, centroids):
    b = p0.shape[0]
    h, w = p0.shape[2], p0.shape[3]
    hw = h * w
    c = centroids.shape[0]
    n = centroids.shape[1]
    c0, c1, c2 = p0.shape[1], p1.shape[1], p2.shape[1]
    hw0 = p0.shape[2] * p0.shape[3]
    hw1 = p1.shape[2] * p1.shape[3]
    hw2 = p2.shape[2] * p2.shape[3]

    bm = 640 if hw > 640 else hw
    nblk = -(-hw // bm)

    m0 = jnp.asarray(_prep_mat(h, w, p0.shape[2], p0.shape[3]), jnp.bfloat16)
    m1 = jnp.asarray(_prep_mat(h, w, p1.shape[2], p1.shape[3]), jnp.bfloat16)
    m2 = jnp.asarray(_prep_mat(h, w, p2.shape[2], p2.shape[3]), jnp.bfloat16)

    xx = (jnp.arange(w, dtype=jnp.float32) / (w - 1)) * 2.0 - 1.0
    yy = (jnp.arange(h, dtype=jnp.float32) / (h - 1)) * 2.0 - 1.0
    gxy = jnp.stack([jnp.tile(xx, h), jnp.repeat(yy, w)])
    w0 = W[:, :c0].astype(jnp.bfloat16)
    w1 = W[:, c0:c0 + c1].astype(jnp.bfloat16)
    w2 = W[:, c0 + c1:c].astype(jnp.bfloat16)
    wxy = W[:, c:c + 2]
    bc = bconv[:, None]
    centb = centroids.astype(jnp.bfloat16)
    centers = jnp.sum(centroids * centroids, axis=0, keepdims=True)

    score = pl.pallas_call(
        _fused_body,
        grid=(b, nblk),
        in_specs=[
            pl.BlockSpec((1, c0, hw0), lambda i, j: (i, 0, 0)),
            pl.BlockSpec((1, c1, hw1), lambda i, j: (i, 0, 0)),
            pl.BlockSpec((1, c2, hw2), lambda i, j: (i, 0, 0)),
            pl.BlockSpec((hw0, bm), lambda i, j: (0, j)),
            pl.BlockSpec((hw1, bm), lambda i, j: (0, j)),
            pl.BlockSpec((hw2, bm), lambda i, j: (0, j)),
            pl.BlockSpec((c, c0), lambda i, j: (0, 0)),
            pl.BlockSpec((c, c1), lambda i, j: (0, 0)),
            pl.BlockSpec((c, c2), lambda i, j: (0, 0)),
            pl.BlockSpec((c, 2), lambda i, j: (0, 0)),
            pl.BlockSpec((c, 1), lambda i, j: (0, 0)),
            pl.BlockSpec((2, bm), lambda i, j: (0, j)),
            pl.BlockSpec((c, n), lambda i, j: (0, 0)),
            pl.BlockSpec((1, n), lambda i, j: (0, 0)),
        ],
        out_specs=pl.BlockSpec((1, 1, bm), lambda i, j: (i * nblk + j, 0, 0)),
        out_shape=jax.ShapeDtypeStruct((b * nblk, 1, bm), jnp.float32),
        scratch_shapes=[pltpu.VMEM((c, hw2), jnp.bfloat16)],
    )(p0.reshape(b, c0, hw0), p1.reshape(b, c1, hw1), p2.reshape(b, c2, hw2),
      m0, m1, m2, w0, w1, w2, wxy, bc, gxy, centb, centers)

    return score.reshape(b, nblk * bm)[:, :hw].reshape(b, 1, h, w)


# speedup vs baseline: 1.8638x; 1.0686x over previous
import functools

import numpy as np

import jax
import jax.numpy as jnp
from jax.experimental import pallas as pl
from jax.experimental.pallas import tpu as pltpu


def _lin_mat(n_out, n_in):
    if n_out == n_in:
        return np.eye(n_in, dtype=np.float64)
    src = (np.arange(n_out) + 0.5) * (n_in / n_out) - 0.5
    i0 = np.floor(src).astype(np.int64)
    f = src - i0
    m = np.zeros((n_out, n_in), dtype=np.float64)
    rows = np.arange(n_out)
    m[rows, np.clip(i0, 0, n_in - 1)] += 1.0 - f
    m[rows, np.clip(i0 + 1, 0, n_in - 1)] += f
    return m


def _pool_mat(n):
    m = np.zeros((n, n), dtype=np.float64)
    idx = np.arange(n)
    for d in (-1, 0, 1):
        j = idx + d
        ok = (j >= 0) & (j < n)
        m[idx[ok], j[ok]] = 1.0
    return m


def _prep_mat(h_out, w_out, h_in, w_in):
    mh = _lin_mat(h_out, h_in) @ _pool_mat(h_in)
    mw = _lin_mat(w_out, w_in) @ _pool_mat(w_in)
    m = np.kron(mh, mw) / 9.0
    return m.T.astype(np.float32)


def _fused_body(p0_ref, p1_ref, p2_ref, m0_ref, m1_ref, m2_ref,
                w0_ref, w1_ref, w2_ref, wxy_ref, bc_ref, gxy_ref,
                cent_ref, centers_ref, out_ref, z2_ref, p0b_ref):
    @pl.when(pl.program_id(1) == 0)
    def _():
        z2_ref[...] = jnp.dot(w2_ref[...], p2_ref[0].astype(jnp.bfloat16),
                              preferred_element_type=jnp.float32
                              ).astype(jnp.bfloat16)
        p0b_ref[...] = p0_ref[0].astype(jnp.bfloat16)
    xp0 = jnp.dot(p0b_ref[...], m0_ref[...],
                  preferred_element_type=jnp.float32).astype(jnp.bfloat16)
    xp1 = jnp.dot(p1_ref[0].astype(jnp.bfloat16), m1_ref[...],
                  preferred_element_type=jnp.float32).astype(jnp.bfloat16)
    ct = (wxy_ref[:, 0:1] * gxy_ref[0:1, :] + wxy_ref[:, 1:2] * gxy_ref[1:2, :]
          + bc_ref[...])
    e = (jnp.dot(w0_ref[...], xp0, preferred_element_type=jnp.float32)
         + jnp.dot(w1_ref[...], xp1, preferred_element_type=jnp.float32)
         + jnp.dot(z2_ref[...], m2_ref[...], preferred_element_type=jnp.float32)
         + ct)
    feats = jnp.sum(e * e, axis=0)[:, None]
    eb = (-2.0 * e).astype(jnp.bfloat16)
    prod = jax.lax.dot_general(eb, cent_ref[...],
                               (((0,), (0,)), ((), ())),
                               preferred_element_type=jnp.float32)
    q = centers_ref[...] + prod

    cur = q
    mins = []
    for _ in range(3):
        mins.append(jnp.min(cur, axis=1))
        cur = jnp.where(cur <= mins[-1][:, None], jnp.inf, cur)
    d0 = jnp.sqrt(jnp.maximum(feats[:, 0] + mins[0], 1e-12))
    d1 = jnp.sqrt(jnp.maximum(feats[:, 0] + mins[1], 1e-12))
    d2s = jnp.sqrt(jnp.maximum(feats[:, 0] + mins[2], 1e-12))
    sm0 = 1.0 / (1.0 + jnp.exp(d0 - d1) + jnp.exp(d0 - d2s))
    out_ref[0, 0] = sm0 * d0


@functools.partial(jax.jit, static_argnums=())
def kernel(p0, p1, p2, W, bconv, centroids):
    b = p0.shape[0]
    h, w = p0.shape[2], p0.shape[3]
    hw = h * w
    c = centroids.shape[0]
    n = centroids.shape[1]
    c0, c1, c2 = p0.shape[1], p1.shape[1], p2.shape[1]
    hw0 = p0.shape[2] * p0.shape[3]
    hw1 = p1.shape[2] * p1.shape[3]
    hw2 = p2.shape[2] * p2.shape[3]

    bm = 640 if hw > 640 else hw
    nblk = -(-hw // bm)

    m0 = jnp.asarray(_prep_mat(h, w, p0.shape[2], p0.shape[3]), jnp.bfloat16)
    m1 = jnp.asarray(_prep_mat(h, w, p1.shape[2], p1.shape[3]), jnp.bfloat16)
    m2 = jnp.asarray(_prep_mat(h, w, p2.shape[2], p2.shape[3]), jnp.bfloat16)

    xx = (jnp.arange(w, dtype=jnp.float32) / (w - 1)) * 2.0 - 1.0
    yy = (jnp.arange(h, dtype=jnp.float32) / (h - 1)) * 2.0 - 1.0
    gxy = jnp.stack([jnp.tile(xx, h), jnp.repeat(yy, w)])
    w0 = W[:, :c0].astype(jnp.bfloat16)
    w1 = W[:, c0:c0 + c1].astype(jnp.bfloat16)
    w2 = W[:, c0 + c1:c].astype(jnp.bfloat16)
    wxy = W[:, c:c + 2]
    bc = bconv[:, None]
    centb = centroids.astype(jnp.bfloat16)
    centers = jnp.sum(centroids * centroids, axis=0, keepdims=True)

    score = pl.pallas_call(
        _fused_body,
        grid=(b, nblk),
        in_specs=[
            pl.BlockSpec((1, c0, hw0), lambda i, j: (i, 0, 0)),
            pl.BlockSpec((1, c1, hw1), lambda i, j: (i, 0, 0)),
            pl.BlockSpec((1, c2, hw2), lambda i, j: (i, 0, 0)),
            pl.BlockSpec((hw0, bm), lambda i, j: (0, j)),
            pl.BlockSpec((hw1, bm), lambda i, j: (0, j)),
            pl.BlockSpec((hw2, bm), lambda i, j: (0, j)),
            pl.BlockSpec((c, c0), lambda i, j: (0, 0)),
            pl.BlockSpec((c, c1), lambda i, j: (0, 0)),
            pl.BlockSpec((c, c2), lambda i, j: (0, 0)),
            pl.BlockSpec((c, 2), lambda i, j: (0, 0)),
            pl.BlockSpec((c, 1), lambda i, j: (0, 0)),
            pl.BlockSpec((2, bm), lambda i, j: (0, j)),
            pl.BlockSpec((c, n), lambda i, j: (0, 0)),
            pl.BlockSpec((1, n), lambda i, j: (0, 0)),
        ],
        out_specs=pl.BlockSpec((1, 1, bm), lambda i, j: (i * nblk + j, 0, 0)),
        out_shape=jax.ShapeDtypeStruct((b * nblk, 1, bm), jnp.float32),
        scratch_shapes=[pltpu.VMEM((c, hw2), jnp.bfloat16),
                        pltpu.VMEM((c0, hw0), jnp.bfloat16)],
    )(p0.reshape(b, c0, hw0), p1.reshape(b, c1, hw1), p2.reshape(b, c2, hw2),
      m0, m1, m2, w0, w1, w2, wxy, bc, gxy, centb, centers)

    return score.reshape(b, nblk * bm)[:, :hw].reshape(b, 1, h, w)


# speedup vs baseline: 1.8978x; 1.0182x over previous
import functools

import numpy as np

import jax
import jax.numpy as jnp
from jax.experimental import pallas as pl
from jax.experimental.pallas import tpu as pltpu


def _lin_mat(n_out, n_in):
    if n_out == n_in:
        return np.eye(n_in, dtype=np.float64)
    src = (np.arange(n_out) + 0.5) * (n_in / n_out) - 0.5
    i0 = np.floor(src).astype(np.int64)
    f = src - i0
    m = np.zeros((n_out, n_in), dtype=np.float64)
    rows = np.arange(n_out)
    m[rows, np.clip(i0, 0, n_in - 1)] += 1.0 - f
    m[rows, np.clip(i0 + 1, 0, n_in - 1)] += f
    return m


def _pool_mat(n):
    m = np.zeros((n, n), dtype=np.float64)
    idx = np.arange(n)
    for d in (-1, 0, 1):
        j = idx + d
        ok = (j >= 0) & (j < n)
        m[idx[ok], j[ok]] = 1.0
    return m


def _prep_mat(h_out, w_out, h_in, w_in):
    mh = _lin_mat(h_out, h_in) @ _pool_mat(h_in)
    mw = _lin_mat(w_out, w_in) @ _pool_mat(w_in)
    m = np.kron(mh, mw) / 9.0
    return m.T.astype(np.float32)


def _fused_body(p0_ref, p1_ref, p2_ref, m0_ref, m1_ref, m2_ref,
                w0_ref, w1_ref, w2_ref, wxy_ref, gxy_ref,
                cent_ref, centers_ref, out_ref, z2_ref, p0b_ref):
    @pl.when(pl.program_id(1) == 0)
    def _():
        z2_ref[...] = jnp.dot(w2_ref[...], p2_ref[0].astype(jnp.bfloat16),
                              preferred_element_type=jnp.float32
                              ).astype(jnp.bfloat16)
        p0b_ref[...] = p0_ref[0].astype(jnp.bfloat16)
    xp0 = jnp.dot(p0b_ref[...], m0_ref[...],
                  preferred_element_type=jnp.float32).astype(jnp.bfloat16)
    xp1 = jnp.dot(p1_ref[0].astype(jnp.bfloat16), m1_ref[...],
                  preferred_element_type=jnp.float32).astype(jnp.bfloat16)
    e = (jnp.dot(w0_ref[...], xp0, preferred_element_type=jnp.float32)
         + jnp.dot(w1_ref[...], xp1, preferred_element_type=jnp.float32)
         + jnp.dot(z2_ref[...], m2_ref[...], preferred_element_type=jnp.float32)
         + jnp.dot(wxy_ref[...], gxy_ref[...],
                   preferred_element_type=jnp.float32))
    feats = jnp.sum(e * e, axis=0)[:, None]
    eb = (-2.0 * e).astype(jnp.bfloat16)
    prod = jax.lax.dot_general(eb, cent_ref[...],
                               (((0,), (0,)), ((), ())),
                               preferred_element_type=jnp.float32)
    q = centers_ref[...] + prod

    cur = q
    mins = []
    for _ in range(3):
        mins.append(jnp.min(cur, axis=1))
        cur = jnp.where(cur <= mins[-1][:, None], jnp.inf, cur)
    d0 = jnp.sqrt(jnp.maximum(feats[:, 0] + mins[0], 1e-12))
    d1 = jnp.sqrt(jnp.maximum(feats[:, 0] + mins[1], 1e-12))
    d2s = jnp.sqrt(jnp.maximum(feats[:, 0] + mins[2], 1e-12))
    sm0 = 1.0 / (1.0 + jnp.exp(d0 - d1) + jnp.exp(d0 - d2s))
    out_ref[0, 0] = sm0 * d0


@functools.partial(jax.jit, static_argnums=())
def kernel(p0, p1, p2, W, bconv, centroids):
    b = p0.shape[0]
    h, w = p0.shape[2], p0.shape[3]
    hw = h * w
    c = centroids.shape[0]
    n = centroids.shape[1]
    c0, c1, c2 = p0.shape[1], p1.shape[1], p2.shape[1]
    hw0 = p0.shape[2] * p0.shape[3]
    hw1 = p1.shape[2] * p1.shape[3]
    hw2 = p2.shape[2] * p2.shape[3]

    bm = 640 if hw > 640 else hw
    nblk = -(-hw // bm)

    m0 = jnp.asarray(_prep_mat(h, w, p0.shape[2], p0.shape[3]), jnp.bfloat16)
    m1 = jnp.asarray(_prep_mat(h, w, p1.shape[2], p1.shape[3]), jnp.bfloat16)
    m2 = jnp.asarray(_prep_mat(h, w, p2.shape[2], p2.shape[3]), jnp.bfloat16)

    xx = (jnp.arange(w, dtype=jnp.float32) / (w - 1)) * 2.0 - 1.0
    yy = (jnp.arange(h, dtype=jnp.float32) / (h - 1)) * 2.0 - 1.0
    gxy = jnp.stack([jnp.tile(xx, h), jnp.repeat(yy, w),
                     jnp.ones((hw,), jnp.float32)]).astype(jnp.bfloat16)
    w0 = W[:, :c0].astype(jnp.bfloat16)
    w1 = W[:, c0:c0 + c1].astype(jnp.bfloat16)
    w2 = W[:, c0 + c1:c].astype(jnp.bfloat16)
    wxy = jnp.concatenate([W[:, c:c + 2], bconv[:, None]],
                          axis=1).astype(jnp.bfloat16)
    centb = centroids.astype(jnp.bfloat16)
    centers = jnp.sum(centroids * centroids, axis=0, keepdims=True)

    score = pl.pallas_call(
        _fused_body,
        grid=(b, nblk),
        in_specs=[
            pl.BlockSpec((1, c0, hw0), lambda i, j: (i, 0, 0)),
            pl.BlockSpec((1, c1, hw1), lambda i, j: (i, 0, 0)),
            pl.BlockSpec((1, c2, hw2), lambda i, j: (i, 0, 0)),
            pl.BlockSpec((hw0, bm), lambda i, j: (0, j)),
            pl.BlockSpec((hw1, bm), lambda i, j: (0, j)),
            pl.BlockSpec((hw2, bm), lambda i, j: (0, j)),
            pl.BlockSpec((c, c0), lambda i, j: (0, 0)),
            pl.BlockSpec((c, c1), lambda i, j: (0, 0)),
            pl.BlockSpec((c, c2), lambda i, j: (0, 0)),
            pl.BlockSpec((c, 3), lambda i, j: (0, 0)),
            pl.BlockSpec((3, bm), lambda i, j: (0, j)),
            pl.BlockSpec((c, n), lambda i, j: (0, 0)),
            pl.BlockSpec((1, n), lambda i, j: (0, 0)),
        ],
        out_specs=pl.BlockSpec((1, 1, bm), lambda i, j: (i * nblk + j, 0, 0)),
        out_shape=jax.ShapeDtypeStruct((b * nblk, 1, bm), jnp.float32),
        scratch_shapes=[pltpu.VMEM((c, hw2), jnp.bfloat16),
                        pltpu.VMEM((c0, hw0), jnp.bfloat16)],
    )(p0.reshape(b, c0, hw0), p1.reshape(b, c1, hw1), p2.reshape(b, c2, hw2),
      m0, m1, m2, w0, w1, w2, wxy, gxy, centb, centers)

    return score.reshape(b, nblk * bm)[:, :hw].reshape(b, 1, h, w)
